# baseline (speedup 1.0000x reference)
import functools

import jax
import jax.numpy as jnp
from jax import lax
from jax.experimental import pallas as pl
from jax.experimental.pallas import tpu as pltpu
from jax.experimental.pallas import tpu_sc as plsc

VOCAB = 2048
HIDDEN = 2048
BATCH = 4
SEQ = 2048
NTOK = BATCH * SEQ
HALF = HIDDEN // 2


def _table_body(e_ref, w_ref, b_ref, t_ref):
    t_ref[...] = lax.dot_general(
        e_ref[...].astype(jnp.bfloat16), w_ref[...].astype(jnp.bfloat16),
        dimension_numbers=(((1,), (1,)), ((), ())),
        preferred_element_type=jnp.float32,
    ) + b_ref[...]


def _build_table_half(emb_weight, lin_weight, lin_bias2d, half):
    BV = 512
    grid = (VOCAB // BV,)
    return pl.pallas_call(
        _table_body,
        grid=grid,
        in_specs=[
            pl.BlockSpec((BV, VOCAB), lambda i: (i, 0)),
            pl.BlockSpec((HALF, VOCAB), lambda i: (half, 0)),
            pl.BlockSpec((1, HALF), lambda i: (0, half)),
        ],
        out_specs=pl.BlockSpec((BV, HALF), lambda i: (i, 0)),
        out_shape=jax.ShapeDtypeStruct((VOCAB, HALF), jnp.float32),
    )(emb_weight, lin_weight, lin_bias2d)


def _make_gather_half(col_off):
    info = plsc.get_sparse_core_info()
    nc, ns = info.num_cores, info.num_subcores
    nw = nc * ns
    b_per_w = NTOK // nw
    chunk = 32
    nchunk = b_per_w // chunk
    mesh = plsc.VectorSubcoreMesh(core_axis_name="c", subcore_axis_name="s")

    @functools.partial(
        pl.kernel, mesh=mesh,
        out_type=(),
        scratch_types=[
            pltpu.VMEM((b_per_w,), jnp.int32),
            pltpu.VMEM((chunk, HALF), jnp.float32),
            pltpu.VMEM((chunk, HALF), jnp.float32),
            pltpu.SemaphoreType.DMA,
            pltpu.SemaphoreType.DMA,
        ],
    )
    def gather(table_hbm, idx_hbm, out_hbm, idx_v, buf0, buf1, sem0, sem1):
        wid = lax.axis_index("s") * nc + lax.axis_index("c")
        base = wid * b_per_w
        pltpu.sync_copy(idx_hbm.at[pl.ds(base, b_per_w)], idx_v)
        bufs = (buf0, buf1)
        sems = (sem0, sem1)
        g = [None, None]
        g[0] = pltpu.async_copy(
            table_hbm.at[idx_v.at[pl.ds(0, chunk)]], bufs[0], sems[0])
        for c in range(nchunk):
            cur = c % 2
            nxt = (c + 1) % 2
            if c + 1 < nchunk:
                g[nxt] = pltpu.async_copy(
                    table_hbm.at[idx_v.at[pl.ds((c + 1) * chunk, chunk)]],
                    bufs[nxt], sems[nxt])
            g[cur].wait()
            pltpu.sync_copy(
                bufs[cur],
                out_hbm.at[pl.ds(base + c * chunk, chunk),
                           pl.ds(col_off, HALF)])

    return gather


_gather0 = _make_gather_half(0)
_gather1 = _make_gather_half(HALF)


def kernel(x, emb_weight, lin_weight, lin_bias):
    bias2d = lin_bias.reshape(1, HIDDEN)
    idx = x.reshape(-1).astype(jnp.int32)
    t0 = _build_table_half(emb_weight, lin_weight, bias2d, 0)
    t1 = _build_table_half(emb_weight, lin_weight, bias2d, 1)
    out_ref = jax.new_ref(jnp.zeros((NTOK, HIDDEN), jnp.float32))
    _gather0(t0, idx, out_ref)
    _gather1(t1, idx, out_ref)
    return out_ref[...].reshape(BATCH, SEQ, HIDDEN)


# speedup vs baseline: 1.2805x; 1.2805x over previous
import functools

import jax
import jax.numpy as jnp
from jax import lax
from jax.experimental import pallas as pl
from jax.experimental.pallas import tpu as pltpu
from jax.experimental.pallas import tpu_sc as plsc

VOCAB = 2048
HIDDEN = 2048
BATCH = 4
SEQ = 2048
NTOK = BATCH * SEQ


def _table_body(e_ref, w_ref, b_ref, t_ref):
    t_ref[...] = lax.dot_general(
        e_ref[...].astype(jnp.bfloat16), w_ref[...].astype(jnp.bfloat16),
        dimension_numbers=(((1,), (1,)), ((), ())),
        preferred_element_type=jnp.float32,
    ) + b_ref[...]


def _build_table(emb_weight, lin_weight, lin_bias):
    BV = 512
    grid = (VOCAB // BV,)
    return pl.pallas_call(
        _table_body,
        grid=grid,
        in_specs=[
            pl.BlockSpec((BV, VOCAB), lambda i: (i, 0)),
            pl.BlockSpec((HIDDEN, VOCAB), lambda i: (0, 0)),
            pl.BlockSpec((1, HIDDEN), lambda i: (0, 0)),
        ],
        out_specs=pl.BlockSpec((BV, HIDDEN), lambda i: (i, 0)),
        out_shape=jax.ShapeDtypeStruct((VOCAB, HIDDEN), jnp.float32),
    )(emb_weight, lin_weight, lin_bias.reshape(1, HIDDEN))


def _make_gather():
    info = plsc.get_sparse_core_info()
    nc, ns = info.num_cores, info.num_subcores
    nw = nc * ns
    b_per_w = NTOK // nw
    chunk = 16
    nbuf = 3
    nchunk = b_per_w // chunk
    mesh = plsc.VectorSubcoreMesh(core_axis_name="c", subcore_axis_name="s")

    @functools.partial(
        pl.kernel, mesh=mesh,
        out_type=jax.ShapeDtypeStruct((NTOK, HIDDEN), jnp.float32),
        scratch_types=[
            pltpu.VMEM((b_per_w,), jnp.int32),
            [pltpu.VMEM((chunk, HIDDEN), jnp.float32) for _ in range(nbuf)],
            [pltpu.SemaphoreType.DMA for _ in range(nbuf)],
        ],
    )
    def gather(table_hbm, idx_hbm, out_hbm, idx_v, bufs, sems):
        wid = lax.axis_index("s") * nc + lax.axis_index("c")
        base = wid * b_per_w
        pltpu.sync_copy(idx_hbm.at[pl.ds(base, b_per_w)], idx_v)
        g = [None] * nbuf
        for j in range(nbuf - 1):
            g[j] = pltpu.async_copy(
                table_hbm.at[idx_v.at[pl.ds(j * chunk, chunk)]],
                bufs[j], sems[j])
        for c in range(nchunk):
            cur = c % nbuf
            if c + nbuf - 1 < nchunk:
                nxt = (c + nbuf - 1) % nbuf
                g[nxt] = pltpu.async_copy(
                    table_hbm.at[idx_v.at[pl.ds((c + nbuf - 1) * chunk, chunk)]],
                    bufs[nxt], sems[nxt])
            g[cur].wait()
            pltpu.sync_copy(bufs[cur], out_hbm.at[pl.ds(base + c * chunk, chunk)])

    return gather


_gather = _make_gather()


def kernel(x, emb_weight, lin_weight, lin_bias):
    table = _build_table(emb_weight, lin_weight, lin_bias)
    idx = x.reshape(-1).astype(jnp.int32)
    out = _gather(table, idx)
    return out.reshape(BATCH, SEQ, HIDDEN)
